# baseline (speedup 1.0000x reference)
import jax
import jax.numpy as jnp
from jax import lax
from jax.experimental import pallas as pl
from jax.experimental.pallas import tpu as pltpu
from jax.experimental.pallas import tpu_sc as plsc

_BOUNDS = ((0, 7), (7, 15), (15, 19), (19, 21), (21, 32),
           (32, 37), (37, 41), (41, 76), (76, 85))
_BATCH = 16384
_NCOL = 85
_D = 64
_NG = 9
_NW = 32
_ROWS_PER_W = _BATCH // _NW
_CHUNK = 32
_NCHUNK = _ROWS_PER_W // _CHUNK
_GROWS = _CHUNK * _NG
_L = 16
_IDXN = _ROWS_PER_W * _NG


def _body(x_hbm, tab_hbm, out_hbm, x_v, idx_v, obuf0, obuf1, stab,
          sem_g0, sem_g1, sem_o0, sem_o1):
  obufs = (obuf0, obuf1)
  sems_g = (sem_g0, sem_g1)
  sems_o = (sem_o0, sem_o1)
  sid = lax.axis_index("s")
  wid = sid * 2 + lax.axis_index("c")
  b0 = wid * _ROWS_PER_W
  lane = lax.broadcasted_iota(jnp.int32, (_L,), 0)
  lane9 = lane * _NG

  @pl.when(sid == 0)
  def _stage_table():
    pltpu.sync_copy(tab_hbm, stab)

  pltpu.sync_copy(x_hbm.at[pl.ds(b0, _ROWS_PER_W)], x_v)

  def rowgrp_body(rg, _):
    ridx = rg * _L + lane
    for g, (s, e) in enumerate(_BOUNDS):
      cur = plsc.load_gather(x_v, [ridx, jnp.full((_L,), s, jnp.int32)])
      arg = jnp.full((_L,), s, jnp.int32)
      for c in range(s + 1, e):
        vals = plsc.load_gather(x_v, [ridx, jnp.full((_L,), c, jnp.int32)])
        m = vals > cur
        cur = jnp.where(m, vals, cur)
        arg = jnp.where(m, c, arg)
      plsc.store_scatter(idx_v, [(rg * _L) * _NG + g + lane9], arg)
    return 0

  lax.fori_loop(0, _ROWS_PER_W // _L, rowgrp_body, 0)
  plsc.subcore_barrier()

  def fire_gathers(i, b):
    cps = []
    base = i * _GROWS
    k = 0
    while k < _GROWS:
      n = min(128, _GROWS - k)
      cps.append(pltpu.async_copy(
          stab.at[idx_v.at[pl.ds(base + k, n)]],
          obufs[b].at[pl.ds(k, n)], sems_g[b]))
      k += n
    return cps

  def out_slice(i):
    return out_hbm.at[pl.ds((b0 + i * _CHUNK) * _NG, _GROWS)]

  for cp in fire_gathers(0, 0):
    cp.wait()
  pltpu.async_copy(obufs[0], out_slice(0), sems_o[0])
  for cp in fire_gathers(1, 1):
    cp.wait()
  pltpu.async_copy(obufs[1], out_slice(1), sems_o[1])

  def ring(t, _):
    i2 = 2 * t + 2
    for b in range(2):
      i = i2 + b
      pltpu.make_async_copy(obufs[b], out_slice(i), sems_o[b]).wait()
      for cp in fire_gathers(i, b):
        cp.wait()
      pltpu.async_copy(obufs[b], out_slice(i), sems_o[b])
    return 0

  lax.fori_loop(0, (_NCHUNK - 2) // 2, ring, 0)
  pltpu.make_async_copy(obufs[0], out_slice(0), sems_o[0]).wait()
  pltpu.make_async_copy(obufs[1], out_slice(1), sems_o[1]).wait()


def kernel(x, W_group_weekday, W_group_time, W_group_gender, W_group_camp,
           W_group_grade, W_group_lane, W_group_district, W_group_area,
           W_group_r):
  table = jnp.concatenate(
      (W_group_weekday, W_group_time, W_group_gender, W_group_camp,
       W_group_grade, W_group_lane, W_group_district, W_group_area,
       W_group_r), axis=0)
  mesh = plsc.VectorSubcoreMesh(core_axis_name="c", subcore_axis_name="s")
  f = pl.kernel(
      _body,
      mesh=mesh,
      compiler_params=pltpu.CompilerParams(
          needs_layout_passes=False, use_tc_tiling_on_sc=False),
      out_type=jax.ShapeDtypeStruct((_BATCH * _NG, _D), jnp.float32),
      scratch_types=[
          pltpu.VMEM((_ROWS_PER_W, _NCOL), jnp.float32),
          pltpu.VMEM((_IDXN,), jnp.int32),
          pltpu.VMEM((_GROWS, _D), jnp.float32),
          pltpu.VMEM((_GROWS, _D), jnp.float32),
          pltpu.VMEM_SHARED((_NCOL, _D), jnp.float32),
          pltpu.SemaphoreType.DMA,
          pltpu.SemaphoreType.DMA,
          pltpu.SemaphoreType.DMA,
          pltpu.SemaphoreType.DMA,
      ],
  )
  out = f(x, table)
  return out.reshape(_BATCH, _NG * _D)


# speedup vs baseline: 1.0706x; 1.0706x over previous
import jax
import jax.numpy as jnp
from jax import lax
from jax.experimental import pallas as pl
from jax.experimental.pallas import tpu as pltpu
from jax.experimental.pallas import tpu_sc as plsc

_BOUNDS = ((0, 7), (7, 15), (15, 19), (19, 21), (21, 32),
           (32, 37), (37, 41), (41, 76), (76, 85))
_BATCH = 16384
_NCOL = 85
_D = 64
_NG = 9
_NW = 32
_ROWS_PER_W = _BATCH // _NW
_CHUNK = 32
_NCHUNK = _ROWS_PER_W // _CHUNK
_GROWS = _CHUNK * _NG
_L = 16
_IDXN = _ROWS_PER_W * _NG


def _body(x_hbm, tab_hbm, out_hbm, x_v, idx_v, obuf0, obuf1, stab,
          sem_g0, sem_g1, sem_o0, sem_o1):
  obufs = (obuf0, obuf1)
  sems_g = (sem_g0, sem_g1)
  sems_o = (sem_o0, sem_o1)
  sid = lax.axis_index("s")
  wid = sid * 2 + lax.axis_index("c")
  b0 = wid * _ROWS_PER_W
  lane = lax.broadcasted_iota(jnp.int32, (_L,), 0)
  lane9 = lane * _NG

  @pl.when(sid == 0)
  def _stage_table():
    pltpu.sync_copy(tab_hbm, stab)

  pltpu.sync_copy(x_hbm.at[pl.ds(b0 * _NCOL, _ROWS_PER_W * _NCOL)], x_v)

  def rowgrp_body(rg, _):
    rbase = (rg * _L + lane) * _NCOL
    for g, (s, e) in enumerate(_BOUNDS):
      cur = plsc.load_gather(x_v, [rbase + s])
      arg = jnp.full((_L,), s, jnp.int32)
      for c in range(s + 1, e):
        vals = plsc.load_gather(x_v, [rbase + c])
        m = vals > cur
        cur = jnp.where(m, vals, cur)
        arg = jnp.where(m, c, arg)
      plsc.store_scatter(idx_v, [(rg * _L) * _NG + g + lane9], arg)
    return 0

  lax.fori_loop(0, _ROWS_PER_W // _L, rowgrp_body, 0)
  plsc.subcore_barrier()

  def fire_gathers(i, b):
    cps = []
    base = i * _GROWS
    k = 0
    while k < _GROWS:
      n = min(128, _GROWS - k)
      cps.append(pltpu.async_copy(
          stab.at[idx_v.at[pl.ds(base + k, n)]],
          obufs[b].at[pl.ds(k, n)], sems_g[b]))
      k += n
    return cps

  def out_slice(i):
    return out_hbm.at[pl.ds((b0 + i * _CHUNK) * _NG, _GROWS)]

  for cp in fire_gathers(0, 0):
    cp.wait()
  pltpu.async_copy(obufs[0], out_slice(0), sems_o[0])
  for cp in fire_gathers(1, 1):
    cp.wait()
  pltpu.async_copy(obufs[1], out_slice(1), sems_o[1])

  def ring(t, _):
    i2 = 2 * t + 2
    for b in range(2):
      i = i2 + b
      pltpu.make_async_copy(obufs[b], out_slice(i), sems_o[b]).wait()
      for cp in fire_gathers(i, b):
        cp.wait()
      pltpu.async_copy(obufs[b], out_slice(i), sems_o[b])
    return 0

  lax.fori_loop(0, (_NCHUNK - 2) // 2, ring, 0)
  pltpu.make_async_copy(obufs[0], out_slice(0), sems_o[0]).wait()
  pltpu.make_async_copy(obufs[1], out_slice(1), sems_o[1]).wait()


def kernel(x, W_group_weekday, W_group_time, W_group_gender, W_group_camp,
           W_group_grade, W_group_lane, W_group_district, W_group_area,
           W_group_r):
  table = jnp.concatenate(
      (W_group_weekday, W_group_time, W_group_gender, W_group_camp,
       W_group_grade, W_group_lane, W_group_district, W_group_area,
       W_group_r), axis=0)
  mesh = plsc.VectorSubcoreMesh(core_axis_name="c", subcore_axis_name="s")
  f = pl.kernel(
      _body,
      mesh=mesh,
      compiler_params=pltpu.CompilerParams(
          needs_layout_passes=False, use_tc_tiling_on_sc=False),
      out_type=jax.ShapeDtypeStruct((_BATCH * _NG, _D), jnp.float32),
      scratch_types=[
          pltpu.VMEM((_ROWS_PER_W * _NCOL,), jnp.float32),
          pltpu.VMEM((_IDXN,), jnp.int32),
          pltpu.VMEM((_GROWS, _D), jnp.float32),
          pltpu.VMEM((_GROWS, _D), jnp.float32),
          pltpu.VMEM_SHARED((_NCOL, _D), jnp.float32),
          pltpu.SemaphoreType.DMA,
          pltpu.SemaphoreType.DMA,
          pltpu.SemaphoreType.DMA,
          pltpu.SemaphoreType.DMA,
      ],
  )
  out = f(x.reshape(_BATCH * _NCOL), table)
  return out.reshape(_BATCH, _NG * _D)
